# baseline (speedup 1.0000x reference)
import functools

import jax
import jax.numpy as jnp
from jax import lax
from jax.experimental import pallas as pl
from jax.experimental.pallas import tpu as pltpu
from jax.experimental.pallas import tpu_sc as plsc

_HOTN = 100000
_MEDIAN_N = 200000
_HASH_SIZE = 500000
_D = 64
_B = 4096
_N = 81920

_NC = 2
_NS = 16
_NW = _NC * _NS
_EPW = _N // _NW
_VPW = _EPW // 16
_G = 128
_LIST = _EPW + _G + 16
_ACC_ROWS = _NS * 257
_ZR = 257


def _sc_body(dic, offsets, hot, med, w_h, w_hash, w_med,
             partial_out, inv_out,
             offs_v, hot_v, med_v, dic_v,
             l_i, l_b,
             bagbuf, rowbuf, outbuf, packbuf, invbuf, acc):
    c = lax.axis_index("c")
    s = lax.axis_index("s")
    wid = s * _NC + c

    def _zb(i, _):
        for q in range(_D // 16):
            outbuf[i, pl.ds(q * 16, 16)] = jnp.zeros((16,), jnp.float32)
        return 0
    lax.fori_loop(0, 128, _zb, 0)
    pltpu.sync_copy(outbuf, acc.at[pl.ds(s * _ZR, 128)])
    pltpu.sync_copy(outbuf, acc.at[pl.ds(s * _ZR + 128, 128)])
    pltpu.sync_copy(outbuf.at[pl.ds(0, 1)], acc.at[pl.ds(s * _ZR + 256, 1)])

    pltpu.sync_copy(offsets, offs_v)
    pltpu.sync_copy(hot, hot_v)
    pltpu.sync_copy(med, med_v)
    pltpu.sync_copy(dic.at[pl.ds(wid * _EPW, _EPW)], dic_v)

    plsc.subcore_barrier()

    lane = lax.broadcasted_iota(jnp.int32, (16,), 0)

    def _p1(i, ptrs):
        pos = wid * _EPW + i * 16 + lane
        seg = jnp.zeros((16,), jnp.int32)
        step = 2048
        while step >= 1:
            cand = seg + step
            val = plsc.load_gather(offs_v, [cand])
            seg = jnp.where(val <= pos, cand, seg)
            step //= 2
        hotb = plsc.load_gather(hot_v, [seg]) != 0
        medb = plsc.load_gather(med_v, [seg]) != 0
        d = dic_v[pl.ds(i * 16, 16)]
        masks = (hotb, jnp.logical_not(hotb),
                 jnp.logical_and(jnp.logical_not(hotb), medb))
        mods = (_HOTN, _HASH_SIZE, _MEDIAN_N)
        new_ptrs = []
        for t in range(3):
            r = d % mods[t]
            pr = lax.shift_right_logical(r, 1)
            par = r & 1
            for h in range(2):
                m = jnp.logical_and(masks[t], par == h)
                p = ptrs[t * 2 + h]
                plsc.store_compressed(l_i[t * 2 + h].at[pl.ds(p, 16)], pr, mask=m)
                plsc.store_compressed(l_b[t * 2 + h].at[pl.ds(p, 16)], seg, mask=m)
                new_ptrs.append(p + jnp.sum(m.astype(jnp.int32)))
        return tuple(new_ptrs)

    ns = lax.fori_loop(0, _VPW, _p1, tuple(jnp.int32(0) for _ in range(6)))

    zero16 = jnp.zeros((16,), jnp.int32)
    trash16 = jnp.full((16,), _B, jnp.int32)
    all16 = lane >= 0
    for q in range(6):
        for k in range(_G // 16):
            plsc.store_compressed(l_i[q].at[pl.ds(ns[q] + k * 16, 16)], zero16, mask=all16)
            plsc.store_compressed(l_b[q].at[pl.ds(ns[q] + k * 16, 16)], trash16, mask=all16)

    tabs = (w_h, w_h, w_hash, w_hash, w_med, w_med)

    def _sweep(q):
        li, lb, n = l_i[q], l_b[q], ns[q]
        par = q % 2
        nch = (n + _G - 1) // _G

        def _chunk(g, _):
            for k in range(_G // 16):
                bagbuf[pl.ds(k * 16, 16)] = lb[pl.ds(g * _G + k * 16, 16)]
            pltpu.sync_copy(tabs[q].at[li.at[pl.ds(g * _G, _G)]], rowbuf)
            def _rp(e, _):
                for v in range(_D // 16):
                    packbuf[e, pl.ds(v * 16, 16)] = rowbuf[e, pl.ds(par * _D + v * 16, 16)]
                return 0
            lax.fori_loop(0, _G, _rp, 0)
            pltpu.sync_copy(packbuf, acc.at[bagbuf], add=True)
            return 0
        lax.fori_loop(0, nch, _chunk, 0)

    for q in range(6):
        _sweep(q)

    @pl.when(c == 0)
    def _():
        nb = _B // _NS
        def _inv(i, _):
            base = s * nb + i * 16
            lo = offs_v[pl.ds(base, 16)]
            hi = plsc.load_gather(offs_v, [base + lane + 1],
                                  mask=(base + lane) < (_B - 1))
            hi = jnp.where((base + lane) == (_B - 1), _N, hi)
            cnt = jnp.maximum(hi - lo, 1).astype(jnp.float32)
            invbuf[pl.ds(i * 16, 16)] = 1.0 / cnt
            return 0
        lax.fori_loop(0, nb // 16, _inv, 0)
        pltpu.sync_copy(invbuf, inv_out.at[pl.ds(s * nb, nb)])

    plsc.subcore_barrier()

    for k in range(2):
        r0 = s * 256 + k * 128
        pltpu.sync_copy(acc.at[pl.ds(r0, 128)], packbuf)
        pltpu.sync_copy(packbuf, partial_out.at[c, pl.ds(r0, 128)])


_sc_kernel = functools.partial(
    pl.kernel,
    out_type=(
        jax.ShapeDtypeStruct((_NC, _B, _D), jnp.float32),
        jax.ShapeDtypeStruct((_B,), jnp.float32),
    ),
    mesh=plsc.VectorSubcoreMesh(core_axis_name="c", subcore_axis_name="s",
                                num_cores=_NC, num_subcores=_NS),
    compiler_params=pltpu.CompilerParams(needs_layout_passes=False,
                                         use_tc_tiling_on_sc=False),
    scratch_types=[
        pltpu.VMEM((_B,), jnp.int32),
        pltpu.VMEM((_B,), jnp.int32),
        pltpu.VMEM((_B,), jnp.int32),
        pltpu.VMEM((_EPW,), jnp.int32),
        [pltpu.VMEM((_LIST,), jnp.int32) for _ in range(6)],
        [pltpu.VMEM((_LIST,), jnp.int32) for _ in range(6)],
        pltpu.VMEM((_G,), jnp.int32),
        pltpu.VMEM((_G, 128), jnp.float32),
        pltpu.VMEM((128, _D), jnp.float32),
        pltpu.VMEM((_G, _D), jnp.float32),
        pltpu.VMEM((_B // _NS,), jnp.float32),
        pltpu.VMEM_SHARED((_ACC_ROWS, _D), jnp.float32),
    ],
)(_sc_body)


def _tc_body(p_ref, inv_ref, o_ref):
    o_ref[...] = (p_ref[0] + p_ref[1]) * inv_ref[...]


def _tc_finalize(partial, inv):
    blk = 512
    return pl.pallas_call(
        _tc_body,
        out_shape=jax.ShapeDtypeStruct((_B, _D), jnp.float32),
        grid=(_B // blk,),
        in_specs=[
            pl.BlockSpec((_NC, blk, _D), lambda i: (0, i, 0)),
            pl.BlockSpec((blk, 1), lambda i: (i, 0)),
        ],
        out_specs=pl.BlockSpec((blk, _D), lambda i: (i, 0)),
    )(partial, inv)


@jax.jit
def kernel(dic, offsets, dic_mask, dic_mask_median, weight_h, weight_hash, weight_median):
    hot = dic_mask.reshape(_B).astype(jnp.int32)
    med = dic_mask_median.reshape(_B).astype(jnp.int32)
    w_h2 = weight_h.reshape(_HOTN // 2, 2 * _D)
    w_hash2 = weight_hash.reshape(_HASH_SIZE // 2, 2 * _D)
    w_med2 = weight_median.reshape(_MEDIAN_N // 2, 2 * _D)
    partial, inv = _sc_kernel(dic.astype(jnp.int32), offsets.astype(jnp.int32),
                              hot, med, w_h2, w_hash2, w_med2)
    return _tc_finalize(partial, inv.reshape(_B, 1))


# speedup vs baseline: 1.5104x; 1.5104x over previous
import functools

import jax
import jax.numpy as jnp
from jax import lax
from jax.experimental import pallas as pl
from jax.experimental.pallas import tpu as pltpu
from jax.experimental.pallas import tpu_sc as plsc

_HOTN = 100000
_MEDIAN_N = 200000
_HASH_SIZE = 500000
_D = 64
_B = 4096
_N = 81920

_NC = 2
_NS = 16
_NW = _NC * _NS
_EPW = _N // _NW
_VPW = _EPW // 16
_BPW = _B // _NW
_G = 128
_LIST = _EPW + _G + 16
_ACC_ROWS = _NS * 257
_ZROWS = 257


def _sc_body(dic, offsets, hot, med, w_h, w_hash, w_med,
             partial_out, inv_out,
             offs_v, hot_v, med_v, dic_v,
             lh_i, lh_b, ls_i, ls_b, lm_i, lm_b,
             bagbuf0, bagbuf1, rowbuf0, rowbuf1, zbuf, invbuf,
             sem0, sem1, acc):
    c = lax.axis_index("c")
    s = lax.axis_index("s")
    wid = s * _NC + c

    def _zb(i, _):
        for q in range(_D // 16):
            zbuf[i, pl.ds(q * 16, 16)] = jnp.zeros((16,), jnp.float32)
        return 0
    lax.fori_loop(0, _ZROWS, _zb, 0)
    pltpu.sync_copy(zbuf, acc.at[pl.ds(s * _ZROWS, _ZROWS)])

    pltpu.sync_copy(offsets, offs_v)
    pltpu.sync_copy(hot, hot_v)
    pltpu.sync_copy(med, med_v)
    pltpu.sync_copy(dic.at[pl.ds(wid * _EPW, _EPW)], dic_v)

    plsc.subcore_barrier()

    lane = lax.broadcasted_iota(jnp.int32, (16,), 0)

    def _p1(i, carry):
        ph, ps, pm = carry
        pos = wid * _EPW + i * 16 + lane
        seg = jnp.zeros((16,), jnp.int32)
        step = 2048
        while step >= 1:
            cand = seg + step
            val = plsc.load_gather(offs_v, [cand])
            seg = jnp.where(val <= pos, cand, seg)
            step //= 2
        hotb = plsc.load_gather(hot_v, [seg]) != 0
        medb = plsc.load_gather(med_v, [seg]) != 0
        d = dic_v[pl.ds(i * 16, 16)]
        mask_s = jnp.logical_not(hotb)
        mask_m = jnp.logical_and(mask_s, medb)
        plsc.store_compressed(lh_i.at[pl.ds(ph, 16)], d % _HOTN, mask=hotb)
        plsc.store_compressed(lh_b.at[pl.ds(ph, 16)], seg, mask=hotb)
        plsc.store_compressed(ls_i.at[pl.ds(ps, 16)], d % _HASH_SIZE, mask=mask_s)
        plsc.store_compressed(ls_b.at[pl.ds(ps, 16)], seg, mask=mask_s)
        plsc.store_compressed(lm_i.at[pl.ds(pm, 16)], d % _MEDIAN_N, mask=mask_m)
        plsc.store_compressed(lm_b.at[pl.ds(pm, 16)], seg, mask=mask_m)
        ph = ph + jnp.sum(hotb.astype(jnp.int32))
        ps = ps + jnp.sum(mask_s.astype(jnp.int32))
        pm = pm + jnp.sum(mask_m.astype(jnp.int32))
        return ph, ps, pm

    nh, ns_, nm = lax.fori_loop(0, _VPW, _p1, (jnp.int32(0), jnp.int32(0), jnp.int32(0)))

    zero16 = jnp.zeros((16,), jnp.int32)
    trash16 = jnp.full((16,), _B, jnp.int32)
    all16 = lane >= 0
    for li, lb, n in ((lh_i, lh_b, nh), (ls_i, ls_b, ns_), (lm_i, lm_b, nm)):
        for k in range(_G // 16):
            plsc.store_compressed(li.at[pl.ds(n + k * 16, 16)], zero16, mask=all16)
            plsc.store_compressed(lb.at[pl.ds(n + k * 16, 16)], trash16, mask=all16)

    rows = (rowbuf0, rowbuf1)
    bags = (bagbuf0, bagbuf1)
    sems = (sem0, sem1)

    def _sweep(table, li, lb, n):
        nch = (n + _G - 1) // _G

        @pl.when(nch > 0)
        def _():
            pltpu.async_copy(table.at[li.at[pl.ds(0, _G)]], rows[0], sems[0])

        def _pair(p, _):
            for b in range(2):
                g = p * 2 + b

                @pl.when(g < nch)
                def _(g=g, b=b):
                    @pl.when(g + 1 < nch)
                    def _():
                        pltpu.async_copy(
                            table.at[li.at[pl.ds((g + 1) * _G, _G)]],
                            rows[1 - b], sems[1 - b])
                    for k in range(_G // 16):
                        bags[b][pl.ds(k * 16, 16)] = lb[pl.ds(g * _G + k * 16, 16)]
                    pltpu.make_async_copy(
                        table.at[pl.ds(0, _G)], rows[b], sems[b]).wait()
                    pltpu.sync_copy(rows[b], acc.at[bags[b]], add=True)
            return 0
        lax.fori_loop(0, (nch + 1) // 2, _pair, 0)

    _sweep(w_h, lh_i, lh_b, nh)
    _sweep(w_hash, ls_i, ls_b, ns_)
    _sweep(w_med, lm_i, lm_b, nm)

    @pl.when(c == 0)
    def _():
        nb = _B // _NS
        def _inv(i, _):
            base = s * nb + i * 16
            lo = offs_v[pl.ds(base, 16)]
            hi = plsc.load_gather(offs_v, [base + lane + 1],
                                  mask=(base + lane) < (_B - 1))
            hi = jnp.where((base + lane) == (_B - 1), _N, hi)
            cnt = jnp.maximum(hi - lo, 1).astype(jnp.float32)
            invbuf[pl.ds(i * 16, 16)] = 1.0 / cnt
            return 0
        lax.fori_loop(0, nb // 16, _inv, 0)
        pltpu.sync_copy(invbuf, inv_out.at[pl.ds(s * nb, nb)])

    plsc.subcore_barrier()

    for k in range(2):
        r0 = s * 256 + k * 128
        pltpu.sync_copy(acc.at[pl.ds(r0, 128)], rows[k])
        pltpu.sync_copy(rows[k], partial_out.at[c, pl.ds(r0, 128)])


_sc_kernel = functools.partial(
    pl.kernel,
    out_type=(
        jax.ShapeDtypeStruct((_NC, _B, _D), jnp.float32),
        jax.ShapeDtypeStruct((_B,), jnp.float32),
    ),
    mesh=plsc.VectorSubcoreMesh(core_axis_name="c", subcore_axis_name="s",
                                num_cores=_NC, num_subcores=_NS),
    compiler_params=pltpu.CompilerParams(needs_layout_passes=False,
                                         use_tc_tiling_on_sc=False),
    scratch_types=[
        pltpu.VMEM((_B,), jnp.int32),
        pltpu.VMEM((_B,), jnp.int32),
        pltpu.VMEM((_B,), jnp.int32),
        pltpu.VMEM((_EPW,), jnp.int32),
        pltpu.VMEM((_LIST,), jnp.int32),
        pltpu.VMEM((_LIST,), jnp.int32),
        pltpu.VMEM((_LIST,), jnp.int32),
        pltpu.VMEM((_LIST,), jnp.int32),
        pltpu.VMEM((_LIST,), jnp.int32),
        pltpu.VMEM((_LIST,), jnp.int32),
        pltpu.VMEM((_G,), jnp.int32),
        pltpu.VMEM((_G,), jnp.int32),
        pltpu.VMEM((_G, _D), jnp.float32),
        pltpu.VMEM((_G, _D), jnp.float32),
        pltpu.VMEM((_ZROWS, _D), jnp.float32),
        pltpu.VMEM((_B // _NS,), jnp.float32),
        pltpu.SemaphoreType.DMA,
        pltpu.SemaphoreType.DMA,
        pltpu.VMEM_SHARED((_ACC_ROWS, _D), jnp.float32),
    ],
)(_sc_body)


def _tc_body(p_ref, inv_ref, o_ref):
    o_ref[...] = (p_ref[0] + p_ref[1]) * inv_ref[...]


def _tc_finalize(partial, inv):
    blk = 512
    return pl.pallas_call(
        _tc_body,
        out_shape=jax.ShapeDtypeStruct((_B, _D), jnp.float32),
        grid=(_B // blk,),
        in_specs=[
            pl.BlockSpec((_NC, blk, _D), lambda i: (0, i, 0)),
            pl.BlockSpec((blk, 1), lambda i: (i, 0)),
        ],
        out_specs=pl.BlockSpec((blk, _D), lambda i: (i, 0)),
    )(partial, inv)


@jax.jit
def kernel(dic, offsets, dic_mask, dic_mask_median, weight_h, weight_hash, weight_median):
    hot = dic_mask.reshape(_B).astype(jnp.int32)
    med = dic_mask_median.reshape(_B).astype(jnp.int32)
    partial, inv = _sc_kernel(dic.astype(jnp.int32), offsets.astype(jnp.int32),
                              hot, med, weight_h, weight_hash, weight_median)
    return _tc_finalize(partial, inv.reshape(_B, 1))
